# Initial kernel scaffold; baseline (speedup 1.0000x reference)
import jax
import jax.numpy as jnp
from jax.experimental import pallas as pl


def kernel(pos, x, r, rk, b, w, h, m, s):
    raise NotImplementedError("write your pallas kernel here")


# speedup vs baseline: 30992.6638x; 30992.6638x over previous
import functools

import jax
import jax.numpy as jnp
from jax.experimental import pallas as pl

GS = 128
N = GS * GS
R = 15
K = 2 * R + 1
NB = 1089
NSHIFT = K * K
NPAD = NB - NSHIFT
T = 10.0
PADW = GS + 2 * R


def _weight(d, k, r_ref, rk_ref, b_ref, w_ref):
    sig = jax.nn.sigmoid(-(d - 1.0) * 10.0)
    core = jnp.zeros_like(d)
    for l in range(rk_ref.shape[1]):
        bl = b_ref[k:k + 1, l:l + 1]
        wl = w_ref[k:k + 1, l:l + 1]
        rkl = rk_ref[k:k + 1, l:l + 1]
        rr = r_ref[k:k + 1, 0:1]
        core = core + bl * jnp.exp(-((d / rr - rkl) / wl) ** 2 / 2.0)
    return sig * core


def _body(x_ref, r_ref, rk_ref, b_ref, w_ref, h_ref, m_ref, s_ref, out_ref,
          *, nb_rules):
    xg = x_ref[...]

    xv = jnp.concatenate([xg[GS - R:, :], xg, xg[:R, :]], axis=0)
    xp = jnp.concatenate([xv[:, GS - R:], xv, xv[:, :R]], axis=1)

    G = jnp.concatenate([xp[si:si + GS, :] for si in range(K)], axis=1)

    a3 = jax.lax.broadcasted_iota(jnp.int32, (K, PADW, GS), 1)
    j3 = jax.lax.broadcasted_iota(jnp.int32, (K, PADW, GS), 2)
    t3 = jax.lax.broadcasted_iota(jnp.int32, (K, PADW, GS), 0)
    masks = (a3 - j3 == t3).astype(jnp.float32).reshape(K, PADW * GS)

    ii = (jax.lax.broadcasted_iota(jnp.int32, (K, K), 0) - R).astype(jnp.float32)
    jj = (jax.lax.broadcasted_iota(jnp.int32, (K, K), 1) - R).astype(jnp.float32)
    dist = jnp.sqrt(ii * ii + jj * jj) / R
    d_self = jnp.zeros((1, 1), jnp.float32)
    d_pad = jnp.full((1, 1), jnp.sqrt(2.0 * (GS // 2) ** 2) / R, jnp.float32)

    delta = jnp.zeros_like(xg)
    for k in range(nb_rules):
        wg = _weight(dist, k, r_ref, rk_ref, b_ref, w_ref)
        w0 = _weight(d_self, k, r_ref, rk_ref, b_ref, w_ref)
        wp = _weight(d_pad, k, r_ref, rk_ref, b_ref, w_ref)
        wsum = jnp.sum(wg).reshape(1, 1) + w0 + NPAD * wp
        inv = 1.0 / (wsum * (NB + 1))

        B = jnp.dot(wg, masks, preferred_element_type=jnp.float32)
        B = B.reshape(K * PADW, GS)

        conv = jnp.dot(G, B, preferred_element_type=jnp.float32)
        pot = (conv + w0 * xg + (NPAD * wp) * xg[0:1, 0:1]) * inv

        mk = m_ref[k:k + 1, 0:1]
        sk = s_ref[k:k + 1, 0:1]
        field = jnp.exp(-(pot - mk) ** 2 / (2.0 * sk * sk) - 0.001) * 2.0 - 1.0
        delta = delta + h_ref[k:k + 1, 0:1] * field

    out_ref[...] = jnp.clip(xg + delta * (1.0 / T), 0.0, 1.0)


@jax.jit
def _run(xg, r, rk, b, w, h, m, s):
    nb_rules = r.shape[0]
    body = functools.partial(_body, nb_rules=nb_rules)
    return pl.pallas_call(
        body,
        out_shape=jax.ShapeDtypeStruct((GS, GS), jnp.float32),
    )(xg, r.reshape(-1, 1), rk, b, w,
      h.reshape(-1, 1), m.reshape(-1, 1), s.reshape(-1, 1))


def kernel(pos, x, r, rk, b, w, h, m, s):
    xg = x.reshape(GS, GS)
    out = _run(xg, r, rk, b, w, h, m, s)
    return (pos, out.reshape(N, 1))
